# Initial kernel scaffold; baseline (speedup 1.0000x reference)
import jax
import jax.numpy as jnp
from jax.experimental import pallas as pl


def kernel(x, edge_index, edge_weights, mlp_W, mlp_b, att, Wz, bz, Wr, br, Wh, bh, lzW, lzb, lrW, lrb, lhW, lhb, lin_W, lin_b):
    raise NotImplementedError("write your pallas kernel here")


# speedup vs baseline: 36.5027x; 36.5027x over previous
import functools

import jax
import jax.numpy as jnp
from jax import lax
from jax.experimental import pallas as pl
from jax.experimental.pallas import tpu as pltpu
from jax.experimental.pallas import tpu_sc as plsc

N = 10000
F = 128
T = 12
OUT = 32
E = 320000

NC = 2
NS = 16
L = 16

CW = 128
NCHUNK = 6
KW = 160
EPT = E // NS
NWIN = EPT // KW
RPT = N // NS

DEG_EPT = E // (NC * NS)
DEG_KW = 1000
DEG_NWIN = DEG_EPT // DEG_KW


def _deg_body(dst_hbm, ew_hbm, out_hbm, deg_sp, dst_v, ew_v, ew_v2, zb_v):
    cid = lax.axis_index("c")
    tid = lax.axis_index("s")

    def _z(i, _):
        zb_v[pl.ds(i * L, L)] = jnp.zeros((L,), jnp.float32)
        return 0
    lax.fori_loop(0, 1024 // L, _z, 0)

    @pl.when(tid < 10)
    def _():
        pltpu.sync_copy(zb_v.at[pl.ds(0, 1000)],
                        deg_sp.at[pl.ds(tid * 1000, 1000)])
    plsc.subcore_barrier()

    ebase = cid * (E // NC) + tid * DEG_EPT

    def _win(wi, _):
        e0 = ebase + wi * DEG_KW
        pltpu.sync_copy(dst_hbm.at[pl.ds(e0, DEG_KW)], dst_v)
        pltpu.sync_copy(ew_hbm.at[pl.ds(e0, DEG_KW)], ew_v)
        pltpu.sync_copy(ew_v, deg_sp.at[dst_v], add=True)
        return 0
    lax.fori_loop(0, DEG_NWIN, _win, 0)
    plsc.subcore_barrier()

    @pl.when(tid < 10)
    def _():
        pltpu.sync_copy(deg_sp.at[pl.ds(tid * 1000, 1000)], ew_v2)
        pltpu.sync_copy(ew_v2, out_hbm.at[pl.ds(cid * N + tid * 1000, 1000)])


def _deg_partials(dst, ew):
    mesh = plsc.VectorSubcoreMesh(core_axis_name="c", subcore_axis_name="s",
                                  num_cores=NC, num_subcores=NS)
    return pl.kernel(
        _deg_body,
        out_type=jax.ShapeDtypeStruct((NC * N,), jnp.float32),
        mesh=mesh,
        scratch_types=[
            pltpu.VMEM_SHARED((N,), jnp.float32),
            pltpu.VMEM((DEG_KW,), jnp.int32),
            pltpu.VMEM((DEG_KW,), jnp.float32),
            pltpu.VMEM((1000,), jnp.float32),
            pltpu.VMEM((1024,), jnp.float32),
        ],
    )(dst, ew)


def _prep_body(xt_ref, degt_ref, mlpw_ref, mlpb_ref, w2_ref, up_ref):
    deg = degt_ref[:, 0:1] + degt_ref[:, 1:2] + 1.0
    dinv = lax.rsqrt(deg)
    acc = jnp.zeros((xt_ref.shape[1], T), jnp.float32)
    for p in range(T):
        acc = acc + jnp.dot(xt_ref[p], mlpw_ref[p],
                            preferred_element_type=jnp.float32)
    feat = jax.nn.sigmoid(acc + mlpb_ref[0:1, :])
    for p in range(T):
        xp = xt_ref[p] * feat[:, p:p + 1]
        y = jnp.dot(xp, w2_ref[...], preferred_element_type=jnp.float32)
        up_ref[p // 2, :, pl.ds((p % 2) * 2 * OUT, 2 * OUT)] = dinv * y


def _prep(xt, degt, mlpw, mlpb, w2, blk=1000):
    grid = (N // blk,)
    return pl.pallas_call(
        _prep_body,
        grid=grid,
        in_specs=[
            pl.BlockSpec((T, blk, F), lambda i: (0, i, 0)),
            pl.BlockSpec((blk, 2), lambda i: (i, 0)),
            pl.BlockSpec((T, F, T), lambda i: (0, 0, 0)),
            pl.BlockSpec((1, T), lambda i: (0, 0)),
            pl.BlockSpec((F, 2 * OUT), lambda i: (0, 0)),
        ],
        out_specs=pl.BlockSpec((NCHUNK, blk, CW), lambda i: (0, i, 0)),
        out_shape=jax.ShapeDtypeStruct((NCHUNK, N, CW), jnp.float32),
    )(xt, degt, mlpw, mlpb, w2)


def _spmm_body(up_hbm, src_hbm, dst_hbm, ew_hbm, out_hbm,
               g_sp, src_v, gsrc_v, dst_v, ew_v, rows_v, gsem):
    cid = lax.axis_index("c")
    tid = lax.axis_index("s")
    ebase = tid * EPT

    for pi in range(NCHUNK // NC):
        chunk = cid * (NCHUNK // NC) + pi
        coff = chunk * N
        for k in range(8):
            s = tid + NS * k

            @pl.when(s < 125)
            def _():
                r0 = s * 80
                pltpu.sync_copy(up_hbm.at[pl.ds(coff + r0, 80)],
                                rows_v.at[pl.ds(0, 80)])
                pltpu.sync_copy(rows_v.at[pl.ds(0, 80)],
                                g_sp.at[pl.ds(r0, 80)])
        plsc.subcore_barrier()

        def _win(wi, _):
            e0 = ebase + wi * KW
            pltpu.sync_copy(src_hbm.at[pl.ds(e0, KW)], src_v)
            pltpu.sync_copy(dst_hbm.at[pl.ds(e0, KW)], dst_v)
            pltpu.sync_copy(ew_hbm.at[pl.ds(e0, KW)], ew_v)
            for j in range(KW // L):
                gsrc_v[pl.ds(j * L, L)] = src_v[pl.ds(j * L, L)] + coff
            pltpu.async_copy(up_hbm.at[gsrc_v], rows_v, gsem).wait()

            def _scale(g, _):
                r0 = g * L
                wv = ew_v[pl.ds(r0, L)]
                for i in range(L):
                    w = wv[i]
                    for j in range(CW // L):
                        rows_v[r0 + i, pl.ds(j * L, L)] = (
                            rows_v[r0 + i, pl.ds(j * L, L)] * w)
                return 0
            lax.fori_loop(0, KW // L, _scale, 0)
            pltpu.sync_copy(rows_v, g_sp.at[dst_v], add=True)
            return 0
        lax.fori_loop(0, NWIN, _win, 0)
        plsc.subcore_barrier()

        for k in range(8):
            s = tid + NS * k

            @pl.when(s < 125)
            def _():
                r0 = s * 80
                pltpu.sync_copy(g_sp.at[pl.ds(r0, 80)],
                                rows_v.at[pl.ds(0, 80)])
                pltpu.sync_copy(rows_v.at[pl.ds(0, 80)],
                                out_hbm.at[pl.ds(coff + r0, 80)])


def _spmm(up_flat, src, dst, ew):
    mesh = plsc.VectorSubcoreMesh(core_axis_name="c", subcore_axis_name="s",
                                  num_cores=NC, num_subcores=NS)
    return pl.kernel(
        _spmm_body,
        out_type=jax.ShapeDtypeStruct((NCHUNK * N, CW), jnp.float32),
        mesh=mesh,
        scratch_types=[
            pltpu.VMEM_SHARED((N, CW), jnp.float32),
            pltpu.VMEM((KW,), jnp.int32),
            pltpu.VMEM((KW,), jnp.int32),
            pltpu.VMEM((KW,), jnp.int32),
            pltpu.VMEM((KW,), jnp.float32),
            pltpu.VMEM((KW, CW), jnp.float32),
            pltpu.SemaphoreType.DMA,
        ],
    )(up_flat, src, dst, ew)


def _final_body(su_ref, degt_ref, lzw_ref, lzb_ref, lhw_ref, lhb_ref,
                bz_ref, bh_ref, att_ref, linw_ref, linb_ref, out_ref):
    blk = degt_ref.shape[0]
    deg = degt_ref[:, 0:1] + degt_ref[:, 1:2] + 1.0
    dinv = lax.rsqrt(deg)
    av = att_ref[0:1, :]
    av = av - jnp.max(av, axis=1, keepdims=True)
    ev = jnp.exp(av)
    probs = ev / jnp.sum(ev, axis=1, keepdims=True)
    acc = jnp.zeros((blk, OUT), jnp.float32)
    for p in range(T):
        base = (p % 2) * 2 * OUT
        gz = dinv * su_ref[p // 2, :, pl.ds(base, OUT)] + bz_ref[0:1, :]
        gh = dinv * su_ref[p // 2, :, pl.ds(base + OUT, OUT)] + bh_ref[0:1, :]
        z = jax.nn.sigmoid(
            jnp.dot(gz, lzw_ref[...], preferred_element_type=jnp.float32)
            + lzb_ref[0:1, :])
        ht = jnp.tanh(
            jnp.dot(gh, lhw_ref[...], preferred_element_type=jnp.float32)
            + lhb_ref[0:1, :])
        acc = acc + probs[0:1, p:p + 1] * ((1.0 - z) * ht)
    y = jnp.dot(jax.nn.relu(acc), linw_ref[...],
                preferred_element_type=jnp.float32) + linb_ref[0:1, :]
    out_ref[...] = y


def _final(su, degt, lzw1t, lzb, lhw1t, lhb, bz, bh, att, linwt, linb,
           blk=1000):
    grid = (N // blk,)
    return pl.pallas_call(
        _final_body,
        grid=grid,
        in_specs=[
            pl.BlockSpec((NCHUNK, blk, CW), lambda i: (0, i, 0)),
            pl.BlockSpec((blk, 2), lambda i: (i, 0)),
            pl.BlockSpec((OUT, OUT), lambda i: (0, 0)),
            pl.BlockSpec((1, OUT), lambda i: (0, 0)),
            pl.BlockSpec((OUT, OUT), lambda i: (0, 0)),
            pl.BlockSpec((1, OUT), lambda i: (0, 0)),
            pl.BlockSpec((1, OUT), lambda i: (0, 0)),
            pl.BlockSpec((1, OUT), lambda i: (0, 0)),
            pl.BlockSpec((1, T), lambda i: (0, 0)),
            pl.BlockSpec((OUT, T), lambda i: (0, 0)),
            pl.BlockSpec((1, T), lambda i: (0, 0)),
        ],
        out_specs=pl.BlockSpec((blk, T), lambda i: (i, 0)),
        out_shape=jax.ShapeDtypeStruct((N, T), jnp.float32),
    )(su, degt, lzw1t, lzb, lhw1t, lhb, bz, bh, att, linwt, linb)


def kernel(x, edge_index, edge_weights, mlp_W, mlp_b, att, Wz, bz, Wr, br,
           Wh, bh, lzW, lzb, lrW, lrb, lhW, lhb, lin_W, lin_b):
    src = edge_index[0]
    dst = edge_index[1]

    degp = _deg_partials(dst, edge_weights)
    degt = degp.reshape(NC, N).T

    xt = jnp.transpose(x, (2, 0, 1))
    mlpw = mlp_W.reshape(T, F, T).transpose(2, 1, 0)
    w2 = jnp.concatenate([Wz, Wh], axis=1)
    up = _prep(xt, degt, mlpw, mlp_b.reshape(1, T), w2)

    su_flat = _spmm(up.reshape(NCHUNK * N, CW), src, dst, edge_weights)
    su = su_flat.reshape(NCHUNK, N, CW)

    return _final(su, degt,
                  lzW[:, :OUT].T, lzb.reshape(1, OUT),
                  lhW[:, :OUT].T, lhb.reshape(1, OUT),
                  bz.reshape(1, OUT), bh.reshape(1, OUT),
                  att.reshape(1, T), lin_W.T, lin_b.reshape(1, T))
